# Initial kernel scaffold; baseline (speedup 1.0000x reference)
import jax
import jax.numpy as jnp
from jax.experimental import pallas as pl


def kernel(e, i, W_up, b_up, W0, b0, W1, b1, W2, b2, W_out):
    raise NotImplementedError("write your pallas kernel here")


# speedup vs baseline: 4.9532x; 4.9532x over previous
import jax
import jax.numpy as jnp
from jax import lax
from jax.experimental import pallas as pl
from jax.experimental.pallas import tpu as pltpu
from jax.experimental.pallas import tpu_sc as plsc

HIDDEN = 128
OUT_EMB = 256
OUT = 128
E = 320000
N = 10000

NC = 2
NS = 16
NW = NC * NS

EDGES_PER_W = E // NW
CHUNK = 80
NCHUNK = EDGES_PER_W // CHUNK
RPT = 624
ZR = 104


def _seg_sum_body(e_hbm, idx_hbm, out_hbm, idx_v, ebuf, zbuf, acc_sh):
    c = lax.axis_index("c")
    s = lax.axis_index("s")
    w = c * NS + s

    zero = jnp.zeros((16,), jnp.float32)

    def _z(k, _):
        zbuf[k // 8, pl.ds((k % 8) * 16, 16)] = zero
        return 0

    lax.fori_loop(0, ZR * 8, _z, 0)
    for t in range(RPT // ZR):
        pltpu.sync_copy(zbuf, acc_sh.at[pl.ds(s * RPT + t * ZR, ZR)])

    @pl.when(s == NS - 1)
    def _zero_tail():
        pltpu.sync_copy(zbuf.at[pl.ds(0, 16)], acc_sh.at[pl.ds(NS * RPT, 16)])

    plsc.subcore_barrier()

    pltpu.sync_copy(idx_hbm.at[w], idx_v)

    def _step(j, _):
        pltpu.sync_copy(e_hbm.at[1, w, j], ebuf)
        pltpu.sync_copy(ebuf, acc_sh.at[idx_v.at[j]], add=True)
        return 0

    lax.fori_loop(0, NCHUNK, _step, 0)
    plsc.subcore_barrier()

    pltpu.sync_copy(
        acc_sh.at[pl.ds(s * RPT, RPT)],
        out_hbm.at[c, pl.ds(s * RPT, RPT), :],
    )

    @pl.when(s == NS - 1)
    def _flush_tail():
        pltpu.sync_copy(
            acc_sh.at[pl.ds(NS * RPT, 16)],
            out_hbm.at[c, pl.ds(NS * RPT, 16), :],
        )


@jax.jit
def _segment_sum_sc(e, idx):
    mesh = plsc.VectorSubcoreMesh(
        core_axis_name="c", subcore_axis_name="s", num_cores=NC, num_subcores=NS
    )
    f = pl.kernel(
        _seg_sum_body,
        out_type=jax.ShapeDtypeStruct((NC, N, HIDDEN), jnp.float32),
        mesh=mesh,
        scratch_types=[
            pltpu.VMEM((NCHUNK, CHUNK), jnp.int32),
            pltpu.VMEM((CHUNK, HIDDEN), jnp.float32),
            pltpu.VMEM((ZR, HIDDEN), jnp.float32),
            pltpu.VMEM_SHARED((N, HIDDEN), jnp.float32),
        ],
    )
    return f(e, idx)


def _mlp_body(p_ref, wu_ref, bu_ref, w0_ref, b0_ref, w1_ref, b1_ref, w2_ref,
              b2_ref, wo_ref, o_ref):
    v = p_ref[0] + p_ref[1]
    v = jnp.dot(v, wu_ref[...], preferred_element_type=jnp.float32) + bu_ref[...]
    for w_ref, b_ref in ((w0_ref, b0_ref), (w1_ref, b1_ref), (w2_ref, b2_ref)):
        v = jnp.dot(v, w_ref[...], preferred_element_type=jnp.float32) + b_ref[...]
        v = v * jax.nn.sigmoid(v)
    o_ref[...] = jnp.dot(v, wo_ref[...], preferred_element_type=jnp.float32)


ROW_BLK = 1000


@jax.jit
def _mlp_tc(p, W_up, b_up, W0, b0, W1, b1, W2, b2, W_out):
    full = lambda shape: pl.BlockSpec(shape, lambda i: (0,) * len(shape))
    return pl.pallas_call(
        _mlp_body,
        grid=(N // ROW_BLK,),
        in_specs=[
            pl.BlockSpec((NC, ROW_BLK, HIDDEN), lambda i: (0, i, 0)),
            full((HIDDEN, OUT_EMB)), full((1, OUT_EMB)),
            full((OUT_EMB, OUT_EMB)), full((1, OUT_EMB)),
            full((OUT_EMB, OUT_EMB)), full((1, OUT_EMB)),
            full((OUT_EMB, OUT_EMB)), full((1, OUT_EMB)),
            full((OUT_EMB, OUT)),
        ],
        out_specs=pl.BlockSpec((ROW_BLK, OUT), lambda i: (i, 0)),
        out_shape=jax.ShapeDtypeStruct((N, OUT), jnp.float32),
    )(p, W_up, b_up.reshape(1, -1), W0, b0.reshape(1, -1), W1, b1.reshape(1, -1),
      W2, b2.reshape(1, -1), W_out)


def kernel(e, i, W_up, b_up, W0, b0, W1, b1, W2, b2, W_out):
    e5 = e.reshape(2, NW, NCHUNK, CHUNK, HIDDEN)
    idx = i.astype(jnp.int32).reshape(NW, NCHUNK, CHUNK)
    p = _segment_sum_sc(e5, idx)
    return _mlp_tc(p, W_up, b_up, W0, b0, W1, b1, W2, b2, W_out)


# speedup vs baseline: 7.7425x; 1.5631x over previous
import jax
import jax.numpy as jnp
from jax import lax
from jax.experimental import pallas as pl
from jax.experimental.pallas import tpu as pltpu
from jax.experimental.pallas import tpu_sc as plsc

HIDDEN = 128
OUT_EMB = 256
OUT = 128
E = 320000
N = 10000

NC = 2
NS = 16
NW = NC * NS

EDGES_PER_W = E // NW
CHUNK = 80
NCHUNK = EDGES_PER_W // CHUNK
RPT = 624
ZR = 104


def _seg_sum_body(e_hbm, idx_hbm, out_hbm, idx_v, ebuf0, ebuf1, zbuf, acc_sh,
                  sem_i, sem0, sem1):
    c = lax.axis_index("c")
    s = lax.axis_index("s")
    w = c * NS + s

    idx_cp = pltpu.async_copy(idx_hbm.at[w], idx_v, sem_i)

    zero = jnp.zeros((16,), jnp.float32)

    def _z(k, _):
        zbuf[k // 8, pl.ds((k % 8) * 16, 16)] = zero
        return 0

    lax.fori_loop(0, ZR * 8, _z, 0)
    for t in range(RPT // ZR):
        pltpu.sync_copy(zbuf, acc_sh.at[pl.ds(s * RPT + t * ZR, ZR)])

    @pl.when(s == NS - 1)
    def _zero_tail():
        pltpu.sync_copy(zbuf.at[pl.ds(0, 16)], acc_sh.at[pl.ds(NS * RPT, 16)])

    idx_cp.wait()
    plsc.subcore_barrier()

    bufs = (ebuf0, ebuf1)
    sems = (sem0, sem1)

    def _load(j, b):
        pltpu.async_copy(e_hbm.at[1, w, j], bufs[b], sems[b])

    def _wait(j, b):
        pltpu.make_async_copy(e_hbm.at[1, w, j], bufs[b], sems[b]).wait()

    def _scatter(j, b):
        pltpu.sync_copy(bufs[b], acc_sh.at[idx_v.at[j]], add=True)

    _load(0, 0)
    _load(1, 1)

    PAIRS = (NCHUNK - 3) // 2

    def _step(t, _):
        j = 2 * t
        _wait(j, 0)
        _scatter(j, 0)
        _load(j + 2, 0)
        _wait(j + 1, 1)
        _scatter(j + 1, 1)
        _load(j + 3, 1)
        return 0

    lax.fori_loop(0, PAIRS, _step, 0)

    j0 = 2 * PAIRS
    for k in range(NCHUNK - 2 * PAIRS):
        _wait(j0 + k, k % 2)
        _scatter(j0 + k, k % 2)
        if j0 + k + 2 < NCHUNK:
            _load(j0 + k + 2, k % 2)

    plsc.subcore_barrier()

    pltpu.sync_copy(
        acc_sh.at[pl.ds(s * RPT, RPT)],
        out_hbm.at[c, pl.ds(s * RPT, RPT), :],
    )

    @pl.when(s == NS - 1)
    def _flush_tail():
        pltpu.sync_copy(
            acc_sh.at[pl.ds(NS * RPT, 16)],
            out_hbm.at[c, pl.ds(NS * RPT, 16), :],
        )


@jax.jit
def _segment_sum_sc(e, idx):
    mesh = plsc.VectorSubcoreMesh(
        core_axis_name="c", subcore_axis_name="s", num_cores=NC, num_subcores=NS
    )
    f = pl.kernel(
        _seg_sum_body,
        out_type=jax.ShapeDtypeStruct((NC, N, HIDDEN), jnp.float32),
        mesh=mesh,
        scratch_types=[
            pltpu.VMEM((NCHUNK, CHUNK), jnp.int32),
            pltpu.VMEM((CHUNK, HIDDEN), jnp.float32),
            pltpu.VMEM((CHUNK, HIDDEN), jnp.float32),
            pltpu.VMEM((ZR, HIDDEN), jnp.float32),
            pltpu.VMEM_SHARED((N, HIDDEN), jnp.float32),
            pltpu.SemaphoreType.DMA,
            pltpu.SemaphoreType.DMA,
            pltpu.SemaphoreType.DMA,
        ],
    )
    return f(e, idx)


def _mlp_body(p_ref, wu_ref, bu_ref, w0_ref, b0_ref, w1_ref, b1_ref, w2_ref,
              b2_ref, wo_ref, o_ref):
    v = p_ref[0] + p_ref[1]
    v = jnp.dot(v, wu_ref[...], preferred_element_type=jnp.float32) + bu_ref[...]
    for w_ref, b_ref in ((w0_ref, b0_ref), (w1_ref, b1_ref), (w2_ref, b2_ref)):
        v = jnp.dot(v, w_ref[...], preferred_element_type=jnp.float32) + b_ref[...]
        v = v * jax.nn.sigmoid(v)
    o_ref[...] = jnp.dot(v, wo_ref[...], preferred_element_type=jnp.float32)


ROW_BLK = 1000


@jax.jit
def _mlp_tc(p, W_up, b_up, W0, b0, W1, b1, W2, b2, W_out):
    full = lambda shape: pl.BlockSpec(shape, lambda i: (0,) * len(shape))
    return pl.pallas_call(
        _mlp_body,
        grid=(N // ROW_BLK,),
        in_specs=[
            pl.BlockSpec((NC, ROW_BLK, HIDDEN), lambda i: (0, i, 0)),
            full((HIDDEN, OUT_EMB)), full((1, OUT_EMB)),
            full((OUT_EMB, OUT_EMB)), full((1, OUT_EMB)),
            full((OUT_EMB, OUT_EMB)), full((1, OUT_EMB)),
            full((OUT_EMB, OUT_EMB)), full((1, OUT_EMB)),
            full((OUT_EMB, OUT)),
        ],
        out_specs=pl.BlockSpec((ROW_BLK, OUT), lambda i: (i, 0)),
        out_shape=jax.ShapeDtypeStruct((N, OUT), jnp.float32),
    )(p, W_up, b_up.reshape(1, -1), W0, b0.reshape(1, -1), W1, b1.reshape(1, -1),
      W2, b2.reshape(1, -1), W_out)


def kernel(e, i, W_up, b_up, W0, b0, W1, b1, W2, b2, W_out):
    e5 = e.reshape(2, NW, NCHUNK, CHUNK, HIDDEN)
    idx = i.astype(jnp.int32).reshape(NW, NCHUNK, CHUNK)
    p = _segment_sum_sc(e5, idx)
    return _mlp_tc(p, W_up, b_up, W0, b0, W1, b1, W2, b2, W_out)


# speedup vs baseline: 7.9046x; 1.0209x over previous
import jax
import jax.numpy as jnp
from jax import lax
from jax.experimental import pallas as pl
from jax.experimental.pallas import tpu as pltpu
from jax.experimental.pallas import tpu_sc as plsc

HIDDEN = 128
OUT_EMB = 256
OUT = 128
E = 320000
N = 10000

NC = 2
NS = 16
NW = NC * NS

EDGES_PER_W = E // NW
CHUNK = 80
NCHUNK = EDGES_PER_W // CHUNK
RPT = 624


def _seg_sum_body(e_hbm, idx_hbm, out_hbm, idx_v, ebuf0, ebuf1, ebuf2,
                  acc_sh, sem_i, l0, l1, l2, s0, s1, s2):
    c = lax.axis_index("c")
    s = lax.axis_index("s")
    w = c * NS + s

    idx_cp = pltpu.async_copy(idx_hbm.at[w], idx_v, sem_i)

    zero = jnp.zeros((16,), jnp.float32)

    def _z(k, _):
        ebuf0[k // 8, pl.ds((k % 8) * 16, 16)] = zero
        return 0

    lax.fori_loop(0, CHUNK * 8, _z, 0)
    for t in range(7):
        pltpu.sync_copy(ebuf0, acc_sh.at[pl.ds(s * RPT + t * CHUNK, CHUNK)])
    pltpu.sync_copy(ebuf0.at[pl.ds(0, 64)],
                    acc_sh.at[pl.ds(s * RPT + 7 * CHUNK, 64)])

    @pl.when(s == NS - 1)
    def _zero_tail():
        pltpu.sync_copy(ebuf0.at[pl.ds(0, 16)], acc_sh.at[pl.ds(NS * RPT, 16)])

    idx_cp.wait()
    plsc.subcore_barrier()

    bufs = (ebuf0, ebuf1, ebuf2)
    lsems = (l0, l1, l2)
    ssems = (s0, s1, s2)

    def _load(j, b):
        pltpu.async_copy(e_hbm.at[1, w, j], bufs[b], lsems[b])

    def _lwait(j, b):
        pltpu.make_async_copy(e_hbm.at[1, w, j], bufs[b], lsems[b]).wait()

    def _scat(j, b):
        pltpu.async_copy(bufs[b], acc_sh.at[idx_v.at[j]], ssems[b], add=True)

    def _swait(j, b):
        pltpu.make_async_copy(bufs[b], acc_sh.at[idx_v.at[j]], ssems[b]).wait()

    _load(0, 0)
    _load(1, 1)
    _load(2, 2)
    _lwait(0, 0)
    _scat(0, 0)
    _lwait(1, 1)
    _scat(1, 1)
    _swait(0, 0)
    _load(3, 0)
    _lwait(2, 2)
    _scat(2, 2)

    TRIPS = (NCHUNK - 2 - 3) // 3

    def _step3(t, _):
        j = 3 * t
        for u in range(3):
            jj = j + u
            _swait(jj - 2, (u + 1) % 3)
            _load(jj + 1, (u + 1) % 3)
            _lwait(jj, u)
            _scat(jj, u)
        return 0

    lax.fori_loop(1, TRIPS + 1, _step3, 0)

    for jj in range(3 * (TRIPS + 1), NCHUNK):
        _swait(jj - 2, (jj - 2) % 3)
        if jj + 1 < NCHUNK:
            _load(jj + 1, (jj + 1) % 3)
        _lwait(jj, jj % 3)
        _scat(jj, jj % 3)

    _swait(NCHUNK - 2, (NCHUNK - 2) % 3)
    _swait(NCHUNK - 1, (NCHUNK - 1) % 3)

    plsc.subcore_barrier()

    pltpu.sync_copy(
        acc_sh.at[pl.ds(s * RPT, RPT)],
        out_hbm.at[c, pl.ds(s * RPT, RPT), :],
    )

    @pl.when(s == NS - 1)
    def _flush_tail():
        pltpu.sync_copy(
            acc_sh.at[pl.ds(NS * RPT, 16)],
            out_hbm.at[c, pl.ds(NS * RPT, 16), :],
        )


@jax.jit
def _segment_sum_sc(e, idx):
    mesh = plsc.VectorSubcoreMesh(
        core_axis_name="c", subcore_axis_name="s", num_cores=NC, num_subcores=NS
    )
    f = pl.kernel(
        _seg_sum_body,
        out_type=jax.ShapeDtypeStruct((NC, N, HIDDEN), jnp.float32),
        mesh=mesh,
        scratch_types=[
            pltpu.VMEM((NCHUNK, CHUNK), jnp.int32),
            pltpu.VMEM((CHUNK, HIDDEN), jnp.float32),
            pltpu.VMEM((CHUNK, HIDDEN), jnp.float32),
            pltpu.VMEM((CHUNK, HIDDEN), jnp.float32),
            pltpu.VMEM_SHARED((N, HIDDEN), jnp.float32),
        ] + [pltpu.SemaphoreType.DMA] * 7,
    )
    return f(e, idx)


def _mlp_body(p_ref, wu_ref, bu_ref, w0_ref, b0_ref, w1_ref, b1_ref, w2_ref,
              b2_ref, wo_ref, o_ref):
    v = p_ref[0] + p_ref[1]
    v = jnp.dot(v, wu_ref[...], preferred_element_type=jnp.float32) + bu_ref[...]
    for w_ref, b_ref in ((w0_ref, b0_ref), (w1_ref, b1_ref), (w2_ref, b2_ref)):
        v = jnp.dot(v, w_ref[...], preferred_element_type=jnp.float32) + b_ref[...]
        v = v * jax.nn.sigmoid(v)
    o_ref[...] = jnp.dot(v, wo_ref[...], preferred_element_type=jnp.float32)


ROW_BLK = 1000


@jax.jit
def _mlp_tc(p, W_up, b_up, W0, b0, W1, b1, W2, b2, W_out):
    full = lambda shape: pl.BlockSpec(shape, lambda i: (0,) * len(shape))
    return pl.pallas_call(
        _mlp_body,
        grid=(N // ROW_BLK,),
        in_specs=[
            pl.BlockSpec((NC, ROW_BLK, HIDDEN), lambda i: (0, i, 0)),
            full((HIDDEN, OUT_EMB)), full((1, OUT_EMB)),
            full((OUT_EMB, OUT_EMB)), full((1, OUT_EMB)),
            full((OUT_EMB, OUT_EMB)), full((1, OUT_EMB)),
            full((OUT_EMB, OUT_EMB)), full((1, OUT_EMB)),
            full((OUT_EMB, OUT)),
        ],
        out_specs=pl.BlockSpec((ROW_BLK, OUT), lambda i: (i, 0)),
        out_shape=jax.ShapeDtypeStruct((N, OUT), jnp.float32),
    )(p, W_up, b_up.reshape(1, -1), W0, b0.reshape(1, -1), W1, b1.reshape(1, -1),
      W2, b2.reshape(1, -1), W_out)


def kernel(e, i, W_up, b_up, W0, b0, W1, b1, W2, b2, W_out):
    e5 = e.reshape(2, NW, NCHUNK, CHUNK, HIDDEN)
    idx = i.astype(jnp.int32).reshape(NW, NCHUNK, CHUNK)
    p = _segment_sum_sc(e5, idx)
    return _mlp_tc(p, W_up, b_up, W0, b0, W1, b1, W2, b2, W_out)


# speedup vs baseline: 8.1382x; 1.0296x over previous
import jax
import jax.numpy as jnp
from jax import lax
from jax.experimental import pallas as pl
from jax.experimental.pallas import tpu as pltpu
from jax.experimental.pallas import tpu_sc as plsc

HIDDEN = 128
OUT_EMB = 256
OUT = 128
E = 320000
N = 10000

NC = 2
NS = 16
NW = NC * NS

EDGES_PER_W = E // NW
CHUNK = 80
NCHUNK = EDGES_PER_W // CHUNK
RPT = 624


def _seg_sum_body(e_hbm, idx_hbm, out_hbm, idx_v, ebuf0, ebuf1, ebuf2,
                  acc_sh, sem_i, sem_z, l0, l1, l2, s0, s1, s2):
    c = lax.axis_index("c")
    s = lax.axis_index("s")
    w = c * NS + s

    idx_cp = pltpu.async_copy(idx_hbm.at[w], idx_v, sem_i)
    pltpu.async_copy(e_hbm.at[1, w, 1], ebuf1, l1)
    pltpu.async_copy(e_hbm.at[1, w, 2], ebuf2, l2)

    zero = jnp.zeros((16,), jnp.float32)
    for r in range(CHUNK):
        for q in range(8):
            ebuf0[r, pl.ds(q * 16, 16)] = zero

    zcps = [
        pltpu.async_copy(ebuf0, acc_sh.at[pl.ds(s * RPT + t * CHUNK, CHUNK)],
                         sem_z)
        for t in range(7)
    ]
    zcps.append(
        pltpu.async_copy(ebuf0.at[pl.ds(0, 64)],
                         acc_sh.at[pl.ds(s * RPT + 7 * CHUNK, 64)], sem_z))

    @pl.when(s == NS - 1)
    def _zero_tail():
        pltpu.sync_copy(ebuf0.at[pl.ds(0, 16)], acc_sh.at[pl.ds(NS * RPT, 16)])

    for cp in zcps:
        cp.wait()
    idx_cp.wait()
    plsc.subcore_barrier()

    bufs = (ebuf0, ebuf1, ebuf2)
    lsems = (l0, l1, l2)
    ssems = (s0, s1, s2)

    def _load(j, b):
        pltpu.async_copy(e_hbm.at[1, w, j], bufs[b], lsems[b])

    def _lwait(j, b):
        pltpu.make_async_copy(e_hbm.at[1, w, j], bufs[b], lsems[b]).wait()

    def _scat(j, b):
        pltpu.async_copy(bufs[b], acc_sh.at[idx_v.at[j]], ssems[b], add=True)

    def _swait(j, b):
        pltpu.make_async_copy(bufs[b], acc_sh.at[idx_v.at[j]], ssems[b]).wait()

    _load(0, 0)
    _lwait(0, 0)
    _scat(0, 0)
    _lwait(1, 1)
    _scat(1, 1)
    _swait(0, 0)
    _load(3, 0)
    _lwait(2, 2)
    _scat(2, 2)

    TRIPS = (NCHUNK - 2 - 3) // 3

    def _step3(t, _):
        j = 3 * t
        for u in range(3):
            jj = j + u
            _swait(jj - 2, (u + 1) % 3)
            _load(jj + 1, (u + 1) % 3)
            _lwait(jj, u)
            _scat(jj, u)
        return 0

    lax.fori_loop(1, TRIPS + 1, _step3, 0)

    for jj in range(3 * (TRIPS + 1), NCHUNK):
        _swait(jj - 2, (jj - 2) % 3)
        if jj + 1 < NCHUNK:
            _load(jj + 1, (jj + 1) % 3)
        _lwait(jj, jj % 3)
        _scat(jj, jj % 3)

    _swait(NCHUNK - 2, (NCHUNK - 2) % 3)
    _swait(NCHUNK - 1, (NCHUNK - 1) % 3)

    plsc.subcore_barrier()

    pltpu.sync_copy(
        acc_sh.at[pl.ds(s * RPT, RPT)],
        out_hbm.at[c, pl.ds(s * RPT, RPT), :],
    )

    @pl.when(s == NS - 1)
    def _flush_tail():
        pltpu.sync_copy(
            acc_sh.at[pl.ds(NS * RPT, 16)],
            out_hbm.at[c, pl.ds(NS * RPT, 16), :],
        )


@jax.jit
def _segment_sum_sc(e, idx):
    mesh = plsc.VectorSubcoreMesh(
        core_axis_name="c", subcore_axis_name="s", num_cores=NC, num_subcores=NS
    )
    f = pl.kernel(
        _seg_sum_body,
        out_type=jax.ShapeDtypeStruct((NC, N, HIDDEN), jnp.float32),
        mesh=mesh,
        scratch_types=[
            pltpu.VMEM((NCHUNK, CHUNK), jnp.int32),
            pltpu.VMEM((CHUNK, HIDDEN), jnp.float32),
            pltpu.VMEM((CHUNK, HIDDEN), jnp.float32),
            pltpu.VMEM((CHUNK, HIDDEN), jnp.float32),
            pltpu.VMEM_SHARED((N, HIDDEN), jnp.float32),
        ] + [pltpu.SemaphoreType.DMA] * 8,
    )
    return f(e, idx)


def _mlp_body(p_ref, wu_ref, bu_ref, w0_ref, b0_ref, w1_ref, b1_ref, w2_ref,
              b2_ref, wo_ref, o_ref):
    v = p_ref[0] + p_ref[1]
    v = jnp.dot(v, wu_ref[...], preferred_element_type=jnp.float32) + bu_ref[...]
    for w_ref, b_ref in ((w0_ref, b0_ref), (w1_ref, b1_ref), (w2_ref, b2_ref)):
        v = jnp.dot(v, w_ref[...], preferred_element_type=jnp.float32) + b_ref[...]
        v = v * jax.nn.sigmoid(v)
    o_ref[...] = jnp.dot(v, wo_ref[...], preferred_element_type=jnp.float32)


ROW_BLK = 2000


@jax.jit
def _mlp_tc(p, W_up, b_up, W0, b0, W1, b1, W2, b2, W_out):
    full = lambda shape: pl.BlockSpec(shape, lambda i: (0,) * len(shape))
    return pl.pallas_call(
        _mlp_body,
        grid=(N // ROW_BLK,),
        in_specs=[
            pl.BlockSpec((NC, ROW_BLK, HIDDEN), lambda i: (0, i, 0)),
            full((HIDDEN, OUT_EMB)), full((1, OUT_EMB)),
            full((OUT_EMB, OUT_EMB)), full((1, OUT_EMB)),
            full((OUT_EMB, OUT_EMB)), full((1, OUT_EMB)),
            full((OUT_EMB, OUT_EMB)), full((1, OUT_EMB)),
            full((OUT_EMB, OUT)),
        ],
        out_specs=pl.BlockSpec((ROW_BLK, OUT), lambda i: (i, 0)),
        out_shape=jax.ShapeDtypeStruct((N, OUT), jnp.float32),
    )(p, W_up, b_up.reshape(1, -1), W0, b0.reshape(1, -1), W1, b1.reshape(1, -1),
      W2, b2.reshape(1, -1), W_out)


def kernel(e, i, W_up, b_up, W0, b0, W1, b1, W2, b2, W_out):
    e5 = e.reshape(2, NW, NCHUNK, CHUNK, HIDDEN)
    idx = i.astype(jnp.int32).reshape(NW, NCHUNK, CHUNK)
    p = _segment_sum_sc(e5, idx)
    return _mlp_tc(p, W_up, b_up, W0, b0, W1, b1, W2, b2, W_out)


# speedup vs baseline: 8.1540x; 1.0019x over previous
import jax
import jax.numpy as jnp
from jax import lax
from jax.experimental import pallas as pl
from jax.experimental.pallas import tpu as pltpu
from jax.experimental.pallas import tpu_sc as plsc

HIDDEN = 128
OUT_EMB = 256
OUT = 128
E = 320000
N = 10000

NC = 2
NS = 16
NW = NC * NS

EDGES_PER_W = E // NW
CHUNK = 80
NCHUNK = EDGES_PER_W // CHUNK
RPT = 624


def _seg_sum_body(e_hbm, idx_hbm, out_hbm, idx_v, ebuf0, ebuf1, ebuf2,
                  acc_sh, sem_i, sem_z, l0, l1, l2, s0, s1, s2):
    c = lax.axis_index("c")
    s = lax.axis_index("s")
    w = c * NS + s

    idx_cp = pltpu.async_copy(idx_hbm.at[w], idx_v, sem_i)
    pltpu.async_copy(e_hbm.at[1, w, 1], ebuf1, l1)
    pltpu.async_copy(e_hbm.at[1, w, 2], ebuf2, l2)

    zero = jnp.zeros((16,), jnp.float32)
    for r in range(CHUNK):
        for q in range(8):
            ebuf0[r, pl.ds(q * 16, 16)] = zero

    zcps = [
        pltpu.async_copy(ebuf0, acc_sh.at[pl.ds(s * RPT + t * CHUNK, CHUNK)],
                         sem_z)
        for t in range(7)
    ]
    zcps.append(
        pltpu.async_copy(ebuf0.at[pl.ds(0, 64)],
                         acc_sh.at[pl.ds(s * RPT + 7 * CHUNK, 64)], sem_z))

    @pl.when(s == NS - 1)
    def _zero_tail():
        pltpu.sync_copy(ebuf0.at[pl.ds(0, 16)], acc_sh.at[pl.ds(NS * RPT, 16)])

    for cp in zcps:
        cp.wait()
    idx_cp.wait()
    plsc.subcore_barrier()

    bufs = (ebuf0, ebuf1, ebuf2)
    lsems = (l0, l1, l2)
    ssems = (s0, s1, s2)

    def _load(j, b):
        pltpu.async_copy(e_hbm.at[1, w, j], bufs[b], lsems[b])

    def _lwait(j, b):
        pltpu.make_async_copy(e_hbm.at[1, w, j], bufs[b], lsems[b]).wait()

    def _scat(j, b):
        pltpu.async_copy(bufs[b], acc_sh.at[idx_v.at[j]], ssems[b], add=True)

    def _swait(j, b):
        pltpu.make_async_copy(bufs[b], acc_sh.at[idx_v.at[j]], ssems[b]).wait()

    _load(0, 0)
    _lwait(0, 0)
    _scat(0, 0)
    _lwait(1, 1)
    _scat(1, 1)
    _swait(0, 0)
    _load(3, 0)
    _lwait(2, 2)
    _scat(2, 2)

    TRIPS = (NCHUNK - 2 - 3) // 3

    def _step3(t, _):
        j = 3 * t
        for u in range(3):
            jj = j + u
            _swait(jj - 2, (u + 1) % 3)
            _load(jj + 1, (u + 1) % 3)
            _lwait(jj, u)
            _scat(jj, u)
        return 0

    lax.fori_loop(1, TRIPS + 1, _step3, 0)

    for jj in range(3 * (TRIPS + 1), NCHUNK):
        _swait(jj - 2, (jj - 2) % 3)
        if jj + 1 < NCHUNK:
            _load(jj + 1, (jj + 1) % 3)
        _lwait(jj, jj % 3)
        _scat(jj, jj % 3)

    _swait(NCHUNK - 2, (NCHUNK - 2) % 3)
    _swait(NCHUNK - 1, (NCHUNK - 1) % 3)

    plsc.subcore_barrier()

    pltpu.sync_copy(
        acc_sh.at[pl.ds(s * RPT, RPT)],
        out_hbm.at[c, pl.ds(s * RPT, RPT), :],
    )

    @pl.when(s == NS - 1)
    def _flush_tail():
        pltpu.sync_copy(
            acc_sh.at[pl.ds(NS * RPT, 16)],
            out_hbm.at[c, pl.ds(NS * RPT, 16), :],
        )


@jax.jit
def _segment_sum_sc(e, idx):
    mesh = plsc.VectorSubcoreMesh(
        core_axis_name="c", subcore_axis_name="s", num_cores=NC, num_subcores=NS
    )
    f = pl.kernel(
        _seg_sum_body,
        out_type=jax.ShapeDtypeStruct((NC, N, HIDDEN), jnp.float32),
        mesh=mesh,
        scratch_types=[
            pltpu.VMEM((NCHUNK, CHUNK), jnp.int32),
            pltpu.VMEM((CHUNK, HIDDEN), jnp.float32),
            pltpu.VMEM((CHUNK, HIDDEN), jnp.float32),
            pltpu.VMEM((CHUNK, HIDDEN), jnp.float32),
            pltpu.VMEM_SHARED((N, HIDDEN), jnp.float32),
        ] + [pltpu.SemaphoreType.DMA] * 8,
    )
    return f(e, idx)


def _mlp_body(p_ref, wu_ref, bu_ref, w0_ref, b0_ref, w1_ref, b1_ref, w2_ref,
              b2_ref, wo_ref, o_ref):
    bf = lambda x: x.astype(jnp.bfloat16)
    v = p_ref[0] + p_ref[1]
    v = jnp.dot(bf(v), bf(wu_ref[...]),
                preferred_element_type=jnp.float32) + bu_ref[...]
    for w_ref, b_ref in ((w0_ref, b0_ref), (w1_ref, b1_ref), (w2_ref, b2_ref)):
        v = jnp.dot(bf(v), bf(w_ref[...]),
                    preferred_element_type=jnp.float32) + b_ref[...]
        v = v * jax.nn.sigmoid(v)
    o_ref[...] = jnp.dot(bf(v), bf(wo_ref[...]),
                         preferred_element_type=jnp.float32)


ROW_BLK = 2000


@jax.jit
def _mlp_tc(p, W_up, b_up, W0, b0, W1, b1, W2, b2, W_out):
    full = lambda shape: pl.BlockSpec(shape, lambda i: (0,) * len(shape))
    return pl.pallas_call(
        _mlp_body,
        grid=(N // ROW_BLK,),
        in_specs=[
            pl.BlockSpec((NC, ROW_BLK, HIDDEN), lambda i: (0, i, 0)),
            full((HIDDEN, OUT_EMB)), full((1, OUT_EMB)),
            full((OUT_EMB, OUT_EMB)), full((1, OUT_EMB)),
            full((OUT_EMB, OUT_EMB)), full((1, OUT_EMB)),
            full((OUT_EMB, OUT_EMB)), full((1, OUT_EMB)),
            full((OUT_EMB, OUT)),
        ],
        out_specs=pl.BlockSpec((ROW_BLK, OUT), lambda i: (i, 0)),
        out_shape=jax.ShapeDtypeStruct((N, OUT), jnp.float32),
    )(p, W_up, b_up.reshape(1, -1), W0, b0.reshape(1, -1), W1, b1.reshape(1, -1),
      W2, b2.reshape(1, -1), W_out)


def kernel(e, i, W_up, b_up, W0, b0, W1, b1, W2, b2, W_out):
    e5 = e.reshape(2, NW, NCHUNK, CHUNK, HIDDEN)
    idx = i.astype(jnp.int32).reshape(NW, NCHUNK, CHUNK)
    p = _segment_sum_sc(e5, idx)
    return _mlp_tc(p, W_up, b_up, W0, b0, W1, b1, W2, b2, W_out)
